# baseline (speedup 1.0000x reference)
import jax
import jax.numpy as jnp
from jax import lax
from jax.experimental import pallas as pl
from jax.experimental.pallas import tpu as pltpu
from jax.experimental.pallas import tpu_sc as plsc

_B, _S, _H = 4, 8192, 32
_NW = 32
_CHUNK = _S // _NW
_NSUB = 4
_SUB = _CHUNK // _NSUB


def _copy_body(x_hbm, o_hbm, buf, *sems):
    w = lax.axis_index("s") * 2 + lax.axis_index("c")
    base = w * _CHUNK

    def in_copy(j):
        return pltpu.make_async_copy(
            x_hbm.at[:, pl.ds(base + j * _SUB, _SUB)],
            buf.at[:, pl.ds(j * _SUB, _SUB)],
            sems[j],
        )

    def out_copy(j):
        return pltpu.make_async_copy(
            buf.at[:, pl.ds(j * _SUB, _SUB)],
            o_hbm.at[:, pl.ds(base + j * _SUB, _SUB)],
            sems[_NSUB + j],
        )

    for j in range(_NSUB):
        in_copy(j).start()
    for j in range(_NSUB):
        in_copy(j).wait()
        out_copy(j).start()
    for j in range(_NSUB):
        out_copy(j).wait()


def kernel(inputs_embeds):
    mesh = plsc.VectorSubcoreMesh(core_axis_name="c", subcore_axis_name="s")
    k = pl.kernel(
        _copy_body,
        out_type=jax.ShapeDtypeStruct((_B, _S, _H), jnp.float32),
        mesh=mesh,
        scratch_types=(
            [pltpu.VMEM((_B, _CHUNK, _H), jnp.float32)]
            + [pltpu.SemaphoreType.DMA] * (2 * _NSUB)
        ),
    )
    return k(inputs_embeds)


# speedup vs baseline: 1.5130x; 1.5130x over previous
import jax
import jax.numpy as jnp
from jax.experimental import pallas as pl
from jax.experimental.pallas import tpu as pltpu

_NCHUNKS = 8


def _copy_kernel(x_ref, o_ref, scratch, *sems):
    in_sems = sems[:_NCHUNKS]
    out_sems = sems[_NCHUNKS:]
    seq = x_ref.shape[1]
    chunk = seq // _NCHUNKS

    def in_copy(i):
        sl = pl.ds(i * chunk, chunk)
        return pltpu.make_async_copy(
            x_ref.at[:, sl], scratch.at[:, sl], in_sems[i]
        )

    def out_copy(i):
        sl = pl.ds(i * chunk, chunk)
        return pltpu.make_async_copy(
            scratch.at[:, sl], o_ref.at[:, sl], out_sems[i]
        )

    for i in range(_NCHUNKS):
        in_copy(i).start()
    for i in range(_NCHUNKS):
        in_copy(i).wait()
        out_copy(i).start()
    for i in range(_NCHUNKS):
        out_copy(i).wait()


def kernel(inputs_embeds):
    shape = inputs_embeds.shape
    return pl.pallas_call(
        _copy_kernel,
        in_specs=[pl.BlockSpec(memory_space=pltpu.MemorySpace.HBM)],
        out_specs=pl.BlockSpec(memory_space=pltpu.MemorySpace.HBM),
        out_shape=jax.ShapeDtypeStruct(shape, inputs_embeds.dtype),
        scratch_shapes=(
            [pltpu.VMEM(shape, inputs_embeds.dtype)]
            + [pltpu.SemaphoreType.DMA] * (2 * _NCHUNKS)
        ),
    )(inputs_embeds)
